# baseline (speedup 1.0000x reference)
import functools

import jax
import jax.numpy as jnp
from jax import lax
from jax.experimental import pallas as pl
from jax.experimental.pallas import tpu as pltpu
from jax.experimental.pallas import tpu_sc as plsc

B = 16
A = 4096
FD = 128
L = 16
NK = FD // L
HALF = A // 2
CHUNK = 256
NCHUNK = HALF // CHUNK


def _mol_gather_kernel(nf_hbm, va_hbm, vf_hbm, out_hbm,
                       buf, sc_vmem, row_buf, partner_buf, shared):
    core = lax.axis_index("c")
    sub = lax.axis_index("s")
    b = core * (B // 2) + sub // 2
    half = sub % 2

    pltpu.sync_copy(va_hbm, sc_vmem.at[0, pl.ds(0, L)])
    pltpu.sync_copy(vf_hbm, sc_vmem.at[1, pl.ds(0, L)])
    idx16 = lax.iota(jnp.int32, L)
    va_b = sc_vmem[0, pl.ds(b, L)][0]
    vf_b = sc_vmem[1, pl.ds(b, L)][0]

    rows = jnp.clip(va_b - half * HALF, 0, HALF)

    accs = [jnp.zeros((L,), jnp.float32) for _ in range(NK)]
    for c in range(NCHUNK):
        start_local = c * CHUNK
        m = jnp.clip(rows - start_local, 0, CHUNK)

        @pl.when(m > 0)
        def _():
            pltpu.sync_copy(
                nf_hbm.at[b, pl.ds(half * HALF + start_local, CHUNK), :], buf)

        def body(j, acc):
            return tuple(acc[k] + buf[j, pl.ds(k * L, L)] for k in range(NK))

        accs = list(lax.fori_loop(0, m, body, tuple(accs)))

    for k in range(NK):
        row_buf[pl.ds(k * L, L)] = accs[k]
    pltpu.sync_copy(row_buf, shared.at[sub])
    plsc.subcore_barrier()

    @pl.when(half == 0)
    def _():
        pltpu.sync_copy(shared.at[sub + 1], partner_buf)
        for k in range(NK):
            tot = accs[k] + partner_buf[pl.ds(k * L, L)]
            keep = (idx16 + k * L) < vf_b
            val = jnp.maximum(jnp.where(keep, tot, jnp.float32(0.0)),
                              jnp.float32(0.0))
            row_buf[pl.ds(k * L, L)] = val
        pltpu.sync_copy(row_buf, out_hbm.at[b])


@jax.jit
def _run(node_features, valid_atoms, valid_feats):
    mesh = plsc.VectorSubcoreMesh(core_axis_name="c", subcore_axis_name="s")
    fn = functools.partial(
        pl.kernel,
        mesh=mesh,
        out_type=jax.ShapeDtypeStruct((B, FD), jnp.float32),
        scratch_types=[
            pltpu.VMEM((CHUNK, FD), jnp.float32),
            pltpu.VMEM((2, 2 * L), jnp.int32),
            pltpu.VMEM((FD,), jnp.float32),
            pltpu.VMEM((FD,), jnp.float32),
            pltpu.VMEM_SHARED((L, FD), jnp.float32),
        ],
    )(_mol_gather_kernel)
    return fn(node_features, valid_atoms, valid_feats)


def kernel(node_features, data_slice):
    ds32 = data_slice.astype(jnp.int32)
    valid_atoms = ds32[:, 0]
    valid_feats = ds32[:, 1]
    return _run(node_features, valid_atoms, valid_feats)


# speedup vs baseline: 1.2056x; 1.2056x over previous
import functools

import jax
import jax.numpy as jnp
from jax import lax
from jax.experimental import pallas as pl
from jax.experimental.pallas import tpu as pltpu
from jax.experimental.pallas import tpu_sc as plsc

B = 16
A = 4096
FD = 128
L = 16
NK = FD // L
CHUNK = 256
TOTCHUNK = A // CHUNK
NSLOT = TOTCHUNK // 2
UNROLL = 8


def _mol_gather_kernel(nf_hbm, va_hbm, vf_hbm, out_hbm,
                       buf0, buf1, sc_vmem, row_buf, partner_buf, shared,
                       sem0, sem1):
    core = lax.axis_index("c")
    sub = lax.axis_index("s")
    b = core * (B // 2) + sub // 2
    half = sub % 2

    pltpu.sync_copy(va_hbm, sc_vmem.at[0, pl.ds(0, L)])
    pltpu.sync_copy(vf_hbm, sc_vmem.at[1, pl.ds(0, L)])
    idx16 = lax.iota(jnp.int32, L)
    va_b = sc_vmem[0, pl.ds(b, L)][0]
    vf_b = sc_vmem[1, pl.ds(b, L)][0]

    total_chunks = (va_b + CHUNK - 1) // CHUNK
    nc0 = (total_chunks + 1) // 2
    my_lo = jnp.where(half == 0, 0, nc0)
    my_hi = jnp.where(half == 0, nc0, total_chunks)
    end_row = jnp.minimum(my_hi * CHUNK, va_b)

    bufs = [buf0, buf1]
    sems = [sem0, sem1]

    def dma_start(i):
        g = my_lo + i

        @pl.when(g < my_hi)
        def _():
            pltpu.async_copy(
                nf_hbm.at[b, pl.ds(g * CHUNK, CHUNK), :], bufs[i % 2],
                sems[i % 2])

    def dma_wait(i):
        g = my_lo + i

        @pl.when(g < my_hi)
        def _():
            pltpu.make_async_copy(
                nf_hbm.at[b, pl.ds(g * CHUNK, CHUNK), :], bufs[i % 2],
                sems[i % 2]).wait()

    accs = tuple(jnp.zeros((L,), jnp.float32) for _ in range(NK))
    dma_start(0)
    for i in range(NSLOT):
        if i + 1 < NSLOT:
            dma_start(i + 1)
        dma_wait(i)
        g = my_lo + i
        m = jnp.clip(end_row - g * CHUNK, 0, CHUNK)
        buf = bufs[i % 2]

        def body(it, acc, buf=buf, m=m):
            base = it * UNROLL
            for r in range(UNROLL):
                j = base + r
                keep = j < m
                acc = tuple(
                    acc[k] + jnp.where(keep, buf[j, pl.ds(k * L, L)],
                                       jnp.float32(0.0))
                    for k in range(NK))
            return acc

        ngroups = (m + UNROLL - 1) // UNROLL
        accs = lax.fori_loop(0, ngroups, body, accs)

    for k in range(NK):
        row_buf[pl.ds(k * L, L)] = accs[k]
    pltpu.sync_copy(row_buf, shared.at[sub])
    plsc.subcore_barrier()

    @pl.when(half == 0)
    def _():
        pltpu.sync_copy(shared.at[sub + 1], partner_buf)
        for k in range(NK):
            tot = accs[k] + partner_buf[pl.ds(k * L, L)]
            keep = (idx16 + k * L) < vf_b
            val = jnp.maximum(jnp.where(keep, tot, jnp.float32(0.0)),
                              jnp.float32(0.0))
            row_buf[pl.ds(k * L, L)] = val
        pltpu.sync_copy(row_buf, out_hbm.at[b])


@jax.jit
def _run(node_features, valid_atoms, valid_feats):
    mesh = plsc.VectorSubcoreMesh(core_axis_name="c", subcore_axis_name="s")
    fn = functools.partial(
        pl.kernel,
        mesh=mesh,
        out_type=jax.ShapeDtypeStruct((B, FD), jnp.float32),
        scratch_types=[
            pltpu.VMEM((CHUNK, FD), jnp.float32),
            pltpu.VMEM((CHUNK, FD), jnp.float32),
            pltpu.VMEM((2, 2 * L), jnp.int32),
            pltpu.VMEM((FD,), jnp.float32),
            pltpu.VMEM((FD,), jnp.float32),
            pltpu.VMEM_SHARED((L, FD), jnp.float32),
            pltpu.SemaphoreType.DMA,
            pltpu.SemaphoreType.DMA,
        ],
    )(_mol_gather_kernel)
    return fn(node_features, valid_atoms, valid_feats)


def kernel(node_features, data_slice):
    ds32 = data_slice.astype(jnp.int32)
    valid_atoms = ds32[:, 0]
    valid_feats = ds32[:, 1]
    return _run(node_features, valid_atoms, valid_feats)
